# baseline (speedup 1.0000x reference)
import jax
import jax.numpy as jnp
from jax.experimental import pallas as pl
from jax.experimental.pallas import tpu as pltpu

B, N, DIMS, K, UNITS = 2, 4096, 3, 16, 128
CH = DIMS * 3 + 1
QB = 1024
BIG = 3.0e38


def _fused_kernel(pcq_ref, pcT_ref, tab_ref, wl_ref, bl_ref, ws_ref,
                  bs_ref, wf_ref, bf_ref, out_ref):
    pcq = pcq_ref[0]
    pcT = pcT_ref[0]
    tab = tab_ref[0]
    wl = wl_ref[...]
    A = wl[0:DIMS] + wl[2 * DIMS:3 * DIMS]
    C = wl[DIMS:2 * DIMS] - wl[2 * DIMS:3 * DIMS]
    w9 = wl[3 * DIMS:3 * DIMS + 1]

    d2 = jnp.zeros((QB, N), dtype=jnp.float32)
    for d in range(DIMS):
        diff = pcq[:, d:d + 1] - pcT[d:d + 1, :]
        d2 = d2 + diff * diff

    qA = jnp.dot(pcq, A, preferred_element_type=jnp.float32)
    att = jnp.zeros((QB, UNITS), dtype=jnp.float32)

    def extract(m_prev):
        dm = jnp.where(d2 > m_prev, d2, BIG)
        parts = [dm[:, g * 128:(g + 1) * 128] for g in range(N // 128)]
        while len(parts) > 1:
            parts = [jnp.minimum(parts[i], parts[i + 1])
                     for i in range(0, len(parts), 2)]
        m = jnp.min(parts[0], axis=1, keepdims=True)
        ohb = (d2 <= m) & (d2 > m_prev)
        oh = jnp.where(ohb, 1.0, 0.0).astype(jnp.bfloat16)
        return oh, m

    def dense(oh, m, att):
        g2 = jnp.dot(oh, tab, preferred_element_type=jnp.float32)
        g = g2[:, :UNITS] + g2[:, UNITS:]
        nf = g[:, 0:UNITS // 2]
        np_k = g[:, UNITS // 2:UNITS // 2 + DIMS]
        nrm = jnp.sqrt(m + 1e-12)
        r = qA + jnp.dot(np_k, C, preferred_element_type=jnp.float32) \
            + nrm * w9 + bl_ref[...][None, :]
        r = jnp.maximum(r, 0.0)
        x = jnp.concatenate([nf, r], axis=-1)
        s = jnp.dot(x, ws_ref[...], preferred_element_type=jnp.float32) \
            + bs_ref[...][None, :]
        s = s - jnp.max(s, axis=-1, keepdims=True)
        e = jnp.exp(s)
        s = e / jnp.sum(e, axis=-1, keepdims=True)
        return att + x * s

    oh_p, m_p = extract(jnp.full((QB, 1), -1.0, dtype=jnp.float32))
    for _ in range(K - 1):
        oh_c, m_c = extract(m_p)
        att = dense(oh_p, m_p, att)
        oh_p, m_p = oh_c, m_c
    att = dense(oh_p, m_p, att)

    out = jnp.maximum(
        jnp.dot(att, wf_ref[...], preferred_element_type=jnp.float32)
        + bf_ref[...][None, :], 0.0)
    out_ref[0] = out


def _run(pc, feats, W_loc, b_loc, W_score, b_score, W_final, b_final):
    Bl = pc.shape[0]
    pcT = jnp.transpose(pc, (0, 2, 1))
    table = jnp.concatenate(
        [feats, pc, jnp.zeros((Bl, N, UNITS - UNITS // 2 - DIMS),
                              dtype=jnp.float32)], axis=-1)
    thi = table.astype(jnp.bfloat16)
    tlo = (table - thi.astype(jnp.float32)).astype(jnp.bfloat16)
    tab = jnp.concatenate([thi, tlo], axis=-1)
    grid = (Bl, N // QB)
    return pl.pallas_call(
        _fused_kernel,
        grid=grid,
        in_specs=[
            pl.BlockSpec((1, QB, DIMS), lambda b, i: (b, i, 0)),
            pl.BlockSpec((1, DIMS, N), lambda b, i: (b, 0, 0)),
            pl.BlockSpec((1, N, 2 * UNITS), lambda b, i: (b, 0, 0)),
            pl.BlockSpec((CH, UNITS // 2), lambda b, i: (0, 0)),
            pl.BlockSpec((UNITS // 2,), lambda b, i: (0,)),
            pl.BlockSpec((UNITS, UNITS), lambda b, i: (0, 0)),
            pl.BlockSpec((UNITS,), lambda b, i: (0,)),
            pl.BlockSpec((UNITS, UNITS), lambda b, i: (0, 0)),
            pl.BlockSpec((UNITS,), lambda b, i: (0,)),
        ],
        out_specs=pl.BlockSpec((1, QB, UNITS), lambda b, i: (b, i, 0)),
        compiler_params=pltpu.CompilerParams(
            dimension_semantics=("parallel", "parallel")),
        out_shape=jax.ShapeDtypeStruct((Bl, N, UNITS), jnp.float32),
    )(pc, pcT, tab, W_loc, b_loc, W_score, b_score, W_final, b_final)


def kernel(pc, feats, W_loc, b_loc, W_score, b_score, W_final, b_final):
    return _run(pc, feats, W_loc, b_loc, W_score, b_score, W_final, b_final)


# speedup vs baseline: 1.0553x; 1.0553x over previous
import jax
import jax.numpy as jnp
from jax.experimental import pallas as pl
from jax.experimental.pallas import tpu as pltpu

B, N, DIMS, K, UNITS = 2, 4096, 3, 16, 128
CH = DIMS * 3 + 1
QB = 1024
BIG = 3.0e38


def _fused_kernel(pcq_ref, pcT_ref, tab_ref, wl_ref, bl_ref, ws_ref,
                  bs_ref, wf_ref, bf_ref, out_ref):
    pcq = pcq_ref[0]
    pcT = pcT_ref[0]
    tab = tab_ref[0]
    wl = wl_ref[...]
    A = wl[0:DIMS] + wl[2 * DIMS:3 * DIMS]
    C = wl[DIMS:2 * DIMS] - wl[2 * DIMS:3 * DIMS]
    w9 = wl[3 * DIMS:3 * DIMS + 1]

    d2 = jnp.zeros((QB, N), dtype=jnp.float32)
    for d in range(DIMS):
        diff = pcq[:, d:d + 1] - pcT[d:d + 1, :]
        d2 = d2 + diff * diff

    qA = jnp.dot(pcq, A, preferred_element_type=jnp.float32)
    att = jnp.zeros((QB, UNITS), dtype=jnp.float32)

    def extract(d2):
        parts = [d2[:, g * 128:(g + 1) * 128] for g in range(N // 128)]
        while len(parts) > 1:
            parts = [jnp.minimum(parts[i], parts[i + 1])
                     for i in range(0, len(parts), 2)]
        m = jnp.min(parts[0], axis=1, keepdims=True)
        ohb = d2 <= m
        oh = jnp.where(ohb, 1.0, 0.0).astype(jnp.bfloat16)
        d2 = jnp.where(ohb, BIG, d2)
        return oh, m, d2

    def dense(oh, m, att):
        g2 = jnp.dot(oh, tab, preferred_element_type=jnp.float32)
        g = g2[:, :UNITS] + g2[:, UNITS:]
        nf = g[:, 0:UNITS // 2]
        np_k = g[:, UNITS // 2:UNITS // 2 + DIMS]
        nrm = jnp.sqrt(m + 1e-12)
        r = qA + jnp.dot(np_k, C, preferred_element_type=jnp.float32) \
            + nrm * w9 + bl_ref[...][None, :]
        r = jnp.maximum(r, 0.0)
        x = jnp.concatenate([nf, r], axis=-1)
        s = jnp.dot(x, ws_ref[...], preferred_element_type=jnp.float32) \
            + bs_ref[...][None, :]
        s = s - jnp.max(s, axis=-1, keepdims=True)
        e = jnp.exp(s)
        s = e / jnp.sum(e, axis=-1, keepdims=True)
        return att + x * s

    oh_p, m_p, d2 = extract(d2)
    for _ in range(K - 1):
        oh_c, m_c, d2 = extract(d2)
        att = dense(oh_p, m_p, att)
        oh_p, m_p = oh_c, m_c
    att = dense(oh_p, m_p, att)

    out = jnp.maximum(
        jnp.dot(att, wf_ref[...], preferred_element_type=jnp.float32)
        + bf_ref[...][None, :], 0.0)
    out_ref[0] = out


def _run(pc, feats, W_loc, b_loc, W_score, b_score, W_final, b_final):
    Bl = pc.shape[0]
    pcT = jnp.transpose(pc, (0, 2, 1))
    table = jnp.concatenate(
        [feats, pc, jnp.zeros((Bl, N, UNITS - UNITS // 2 - DIMS),
                              dtype=jnp.float32)], axis=-1)
    thi = table.astype(jnp.bfloat16)
    tlo = (table - thi.astype(jnp.float32)).astype(jnp.bfloat16)
    tab = jnp.concatenate([thi, tlo], axis=-1)
    grid = (Bl, N // QB)
    return pl.pallas_call(
        _fused_kernel,
        grid=grid,
        in_specs=[
            pl.BlockSpec((1, QB, DIMS), lambda b, i: (b, i, 0)),
            pl.BlockSpec((1, DIMS, N), lambda b, i: (b, 0, 0)),
            pl.BlockSpec((1, N, 2 * UNITS), lambda b, i: (b, 0, 0)),
            pl.BlockSpec((CH, UNITS // 2), lambda b, i: (0, 0)),
            pl.BlockSpec((UNITS // 2,), lambda b, i: (0,)),
            pl.BlockSpec((UNITS, UNITS), lambda b, i: (0, 0)),
            pl.BlockSpec((UNITS,), lambda b, i: (0,)),
            pl.BlockSpec((UNITS, UNITS), lambda b, i: (0, 0)),
            pl.BlockSpec((UNITS,), lambda b, i: (0,)),
        ],
        out_specs=pl.BlockSpec((1, QB, UNITS), lambda b, i: (b, i, 0)),
        compiler_params=pltpu.CompilerParams(
            dimension_semantics=("parallel", "parallel")),
        out_shape=jax.ShapeDtypeStruct((Bl, N, UNITS), jnp.float32),
    )(pc, pcT, tab, W_loc, b_loc, W_score, b_score, W_final, b_final)


def kernel(pc, feats, W_loc, b_loc, W_score, b_score, W_final, b_final):
    return _run(pc, feats, W_loc, b_loc, W_score, b_score, W_final, b_final)
